# baseline (speedup 1.0000x reference)
import functools

import jax
import jax.numpy as jnp
from jax import lax
from jax.experimental import pallas as pl
from jax.experimental.pallas import tpu as pltpu
from jax.experimental.pallas import tpu_sc as plsc

BATCH = 16384
EMB = 32
NC = 2
NS = 16
NW = NC * NS
B_PER_W = BATCH // NW
CHUNK = 128
NCHUNK = B_PER_W // CHUNK
T4_ROWS = 250000
LANES = 16


def _sc_gather_body(t4_item, item_idx, t4_user, user_idx,
                    item_out, user_out,
                    idx_v, idx4_v, rem32_v, rows_v, out_v, sem):
    wid = lax.axis_index("s") * NC + lax.axis_index("c")

    for tbl, idx_hbm, out_hbm in (
        (t4_item, item_idx, item_out),
        (t4_user, user_idx, user_out),
    ):
        pltpu.sync_copy(idx_hbm.at[wid], idx_v)

        def split_idx(j, _):
            c = j // (CHUNK // LANES)
            l = (j % (CHUNK // LANES)) * LANES
            raw = idx_v[c, pl.ds(l, LANES)]
            idx4_v[c, pl.ds(l, LANES)] = raw >> 2
            rem32_v[c, pl.ds(l, LANES)] = (raw & 3) << 5
            return 0

        lax.fori_loop(0, NCHUNK * (CHUNK // LANES), split_idx, 0,
                      unroll=True)

        copies = [
            pltpu.async_copy(tbl.at[idx4_v.at[c]],
                             rows_v.at[pl.ds(c * CHUNK, CHUNK)], sem)
            for c in range(NCHUNK)
        ]
        for cp in copies:
            cp.wait()

        def extract(j, _):
            c = j // (CHUNK // LANES)
            l = (j % (CHUNK // LANES)) * LANES
            b0 = c * CHUNK + l
            brow = jax.lax.iota(jnp.int32, LANES) + b0
            col0 = rem32_v[c, pl.ds(l, LANES)]
            for f in range(EMB):
                vals = plsc.load_gather(rows_v, [brow, col0 + f])
                out_v[f, pl.ds(b0, LANES)] = vals
            return 0

        lax.fori_loop(0, NCHUNK * (CHUNK // LANES), extract, 0)

        pltpu.sync_copy(out_v, out_hbm.at[wid])


@functools.cache
def _sc_gather_kernel():
    mesh = plsc.VectorSubcoreMesh(core_axis_name="c", subcore_axis_name="s")
    return pl.kernel(
        _sc_gather_body,
        mesh=mesh,
        compiler_params=pltpu.CompilerParams(needs_layout_passes=False),
        out_type=[
            jax.ShapeDtypeStruct((NW, EMB, B_PER_W), jnp.float32),
            jax.ShapeDtypeStruct((NW, EMB, B_PER_W), jnp.float32),
        ],
        scratch_types=[
            pltpu.VMEM((NCHUNK, CHUNK), jnp.int32),
            pltpu.VMEM((NCHUNK, CHUNK), jnp.int32),
            pltpu.VMEM((NCHUNK, CHUNK), jnp.int32),
            pltpu.VMEM((B_PER_W, CHUNK), jnp.float32),
            pltpu.VMEM((EMB, B_PER_W), jnp.float32),
            pltpu.SemaphoreType.DMA,
        ],
    )


def _mlp_body(item_ref, user_ref, w1a_ref, w1b_ref, b1_ref,
              w2_ref, b2_ref, w3_ref, b3_ref, w4_ref, b4_ref, out_ref):
    xi = item_ref[0]
    xu = user_ref[0]
    h = w1a_ref[...] @ xi + w1b_ref[...] @ xu
    h = jax.nn.relu(h + b1_ref[...])
    h = jax.nn.relu(w2_ref[...] @ h + b2_ref[...])
    h = jax.nn.relu(w3_ref[...] @ h + b3_ref[...])
    o = w4_ref[...] @ h + b4_ref[...]
    out_ref[0] = jax.nn.sigmoid(o)


def _mlp(item_g, user_g, w1a, w1b, b1, w2, b2, w3, b3, w4, b4):
    full = lambda shape: pl.BlockSpec(shape, lambda i: tuple(0 for _ in shape))
    return pl.pallas_call(
        _mlp_body,
        grid=(NW,),
        in_specs=[
            pl.BlockSpec((1, EMB, B_PER_W), lambda i: (i, 0, 0)),
            pl.BlockSpec((1, EMB, B_PER_W), lambda i: (i, 0, 0)),
            full(w1a.shape), full(w1b.shape), full(b1.shape),
            full(w2.shape), full(b2.shape),
            full(w3.shape), full(b3.shape),
            full(w4.shape), full(b4.shape),
        ],
        out_specs=pl.BlockSpec((1, 1, B_PER_W), lambda i: (i, 0, 0)),
        out_shape=jax.ShapeDtypeStruct((NW, 1, B_PER_W), jnp.float32),
    )(item_g, user_g, w1a, w1b, b1, w2, b2, w3, b3, w4, b4)


def kernel(item_input, user_input, emb_item, emb_user,
           W1, b1, W2, b2, W3, b3, W4, b4):
    t4_item = emb_item.reshape(T4_ROWS, 4 * EMB)
    t4_user = emb_user.reshape(T4_ROWS, 4 * EMB)
    item_idx = item_input.astype(jnp.int32).reshape(NW, NCHUNK, CHUNK)
    user_idx = user_input.astype(jnp.int32).reshape(NW, NCHUNK, CHUNK)
    item_g, user_g = _sc_gather_kernel()(
        t4_item, item_idx, t4_user, user_idx)
    w1a = W1[:, :EMB]
    w1b = W1[:, EMB:]
    out = _mlp(item_g, user_g,
               w1a, w1b, b1.reshape(-1, 1),
               W2, b2.reshape(-1, 1),
               W3, b3.reshape(-1, 1),
               W4, b4.reshape(-1, 1))
    return out.reshape(BATCH)


# speedup vs baseline: 1.0246x; 1.0246x over previous
import functools

import jax
import jax.numpy as jnp
from jax import lax
from jax.experimental import pallas as pl
from jax.experimental.pallas import tpu as pltpu
from jax.experimental.pallas import tpu_sc as plsc

BATCH = 16384
EMB = 32
NC = 2
NS = 16
NW = NC * NS
B_PER_W = BATCH // NW
CHUNK = 128
NCHUNK = B_PER_W // CHUNK


def _sc_gather_body(t_item, item_idx, t_user, user_idx,
                    item_out, user_out,
                    iidx_v, uidx_v, irows_v, urows_v, isem, usem):
    wid = lax.axis_index("s") * NC + lax.axis_index("c")
    pltpu.sync_copy(item_idx.at[wid], iidx_v)
    pltpu.sync_copy(user_idx.at[wid], uidx_v)
    copies = []
    for c in range(NCHUNK):
        copies.append(pltpu.async_copy(
            t_item.at[iidx_v.at[c]],
            irows_v.at[pl.ds(c * CHUNK, CHUNK)], isem))
        copies.append(pltpu.async_copy(
            t_user.at[uidx_v.at[c]],
            urows_v.at[pl.ds(c * CHUNK, CHUNK)], usem))
    for cp in copies:
        cp.wait()
    pltpu.sync_copy(irows_v, item_out.at[wid])
    pltpu.sync_copy(urows_v, user_out.at[wid])


@functools.cache
def _sc_gather_kernel():
    mesh = plsc.VectorSubcoreMesh(core_axis_name="c", subcore_axis_name="s")
    return pl.kernel(
        _sc_gather_body,
        mesh=mesh,
        compiler_params=pltpu.CompilerParams(
            needs_layout_passes=False, use_tc_tiling_on_sc=False),
        out_type=[
            jax.ShapeDtypeStruct((NW, B_PER_W, EMB), jnp.float32),
            jax.ShapeDtypeStruct((NW, B_PER_W, EMB), jnp.float32),
        ],
        scratch_types=[
            pltpu.VMEM((NCHUNK, CHUNK), jnp.int32),
            pltpu.VMEM((NCHUNK, CHUNK), jnp.int32),
            pltpu.VMEM((B_PER_W, EMB), jnp.float32),
            pltpu.VMEM((B_PER_W, EMB), jnp.float32),
            pltpu.SemaphoreType.DMA,
            pltpu.SemaphoreType.DMA,
        ],
    )


BLK = 2048


def _mlp_body(item_ref, user_ref, w1a_ref, w1b_ref, b1_ref,
              w2_ref, b2_ref, w3_ref, b3_ref, w4_ref, b4_ref, out_ref):
    h = item_ref[...] @ w1a_ref[...] + user_ref[...] @ w1b_ref[...]
    h = jax.nn.relu(h + b1_ref[...])
    h = jax.nn.relu(h @ w2_ref[...] + b2_ref[...])
    h = jax.nn.relu(h @ w3_ref[...] + b3_ref[...])
    o = h @ w4_ref[...] + b4_ref[...]
    out_ref[...] = jax.nn.sigmoid(o)


def _mlp(item_rows, user_rows, w1a, w1b, b1, w2, b2, w3, b3, w4, b4):
    grid = (BATCH // BLK,)
    full = lambda shape: pl.BlockSpec(shape, lambda i: (0, 0))
    return pl.pallas_call(
        _mlp_body,
        grid=grid,
        in_specs=[
            pl.BlockSpec((BLK, EMB), lambda i: (i, 0)),
            pl.BlockSpec((BLK, EMB), lambda i: (i, 0)),
            full(w1a.shape), full(w1b.shape), full(b1.shape),
            full(w2.shape), full(b2.shape),
            full(w3.shape), full(b3.shape),
            full(w4.shape), full(b4.shape),
        ],
        out_specs=pl.BlockSpec((BLK, 1), lambda i: (i, 0)),
        out_shape=jax.ShapeDtypeStruct((BATCH, 1), jnp.float32),
    )(item_rows, user_rows, w1a, w1b, b1, w2, b2, w3, b3, w4, b4)


def kernel(item_input, user_input, emb_item, emb_user,
           W1, b1, W2, b2, W3, b3, W4, b4):
    item_idx = item_input.astype(jnp.int32).reshape(NW, NCHUNK, CHUNK)
    user_idx = user_input.astype(jnp.int32).reshape(NW, NCHUNK, CHUNK)
    item_g, user_g = _sc_gather_kernel()(
        emb_item, item_idx, emb_user, user_idx)
    out = _mlp(item_g.reshape(BATCH, EMB), user_g.reshape(BATCH, EMB),
               W1[:, :EMB].T, W1[:, EMB:].T, b1.reshape(1, -1),
               W2.T, b2.reshape(1, -1),
               W3.T, b3.reshape(1, -1),
               W4.T, b4.reshape(1, 1))
    return out.reshape(BATCH)


# speedup vs baseline: 1.3758x; 1.3428x over previous
import functools

import jax
import jax.numpy as jnp
from jax import lax
from jax.experimental import pallas as pl
from jax.experimental.pallas import tpu as pltpu
from jax.experimental.pallas import tpu_sc as plsc

BATCH = 16384
EMB = 32
NC = 2
NS = 16
NW = NC * NS
NIDX = 1000000
ALIGNED = 999936
BASE_TC = 244
WIN = 512
NWIN = 61
CAP = 160
DUMP = BATCH
LANES = 16


def _sc_gather_body(t_item, t_user, tail_item, tail_user, idx_all,
                    item_out, user_out,
                    idx_v, li_v, lp_v, wi_v, rows_v, pos_v, win_v,
                    tail_v, sem, ssem):
    wid = lax.axis_index("s") * NC + lax.axis_index("c")
    startc = BASE_TC * wid + jnp.minimum(wid, 4)
    start = startc * 128
    extra = (wid < 4).astype(jnp.int32)
    end_aligned = (startc + BASE_TC + extra) * 128
    scan_end = jnp.where(wid == NW - 1, NIDX, end_aligned)
    iota = lax.iota(jnp.int32, LANES)

    def one_table(tbl, tail_tbl, out_hbm, tbl_pos):
        pltpu.sync_copy(idx_all.at[tbl_pos], idx_v)
        pltpu.sync_copy(tail_tbl, tail_v)

        def scan(j, cnt):
            r = j // 8
            l = (j % 8) * LANES
            raw = idx_v[r, pl.ds(l, LANES)]
            m = (raw >= start) & (raw < scan_end)
            plsc.store_compressed(li_v.at[pl.ds(cnt, LANES)], raw, mask=m)
            pos = j * LANES + iota
            plsc.store_compressed(lp_v.at[pl.ds(cnt, LANES)], pos, mask=m)
            return cnt + plsc.all_reduce_population_count(m)[0]

        count = lax.fori_loop(0, BATCH // LANES, scan, 0)
        ntrip = (count + LANES - 1) // LANES

        def do_window(c0, wsize, from_tail):
            def collect(q, wcnt):
                il = li_v[pl.ds(q * LANES, LANES)]
                ip = lp_v[pl.ds(q * LANES, LANES)]
                valid = (q * LANES + iota) < count
                m = valid & (il >= c0) & (il < c0 + wsize)
                plsc.store_compressed(wi_v.at[pl.ds(wcnt, LANES)], il - c0,
                                      mask=m)
                plsc.store_compressed(wi_v.at[pl.ds(CAP + wcnt, LANES)], ip,
                                      mask=m)
                return wcnt + plsc.all_reduce_population_count(m)[0]

            wcount = lax.fori_loop(0, ntrip, collect, 0)

            @pl.when(wcount > 0)
            def _():
                if not from_tail:
                    pltpu.sync_copy(t_src.at[:, pl.ds(c0, wsize)],
                                    win_v.at[:, pl.ds(0, wsize)])
                src_v = tail_v if from_tail else win_v
                for q in range(8):
                    lane = q * LANES + iota
                    pv = wi_v[pl.ds(CAP + q * LANES, LANES)]
                    pv = jnp.where(lane < wcount, pv,
                                   DUMP + ((wid * 4 + q) & 127))
                    pos_v[0, pl.ds(q * LANES, LANES)] = pv

                def extract(q, _):
                    cols = wi_v[pl.ds(q * LANES, LANES)]
                    for k in range(LANES):
                        col = jnp.clip(cols[k], 0, wsize - 1)
                        csp = jnp.full((LANES,), col, jnp.int32)
                        v0 = plsc.load_gather(src_v, [iota, csp])
                        v1 = plsc.load_gather(src_v, [iota + LANES, csp])
                        m = q * LANES + k
                        rows_v[m, pl.ds(0, LANES)] = v0
                        rows_v[m, pl.ds(LANES, LANES)] = v1
                    return 0

                wtrip = (wcount + LANES - 1) // LANES
                lax.fori_loop(0, wtrip, extract, 0)
                pltpu.async_copy(rows_v, out_hbm.at[pos_v.at[0]], ssem).wait()

        def window_loop(w, _):
            do_window(start + w * WIN, WIN, False)
            return 0

        t_src = tbl
        lax.fori_loop(0, NWIN, window_loop, 0)

        @pl.when(wid < 4)
        def _():
            do_window(end_aligned - 128, 128, False)

        @pl.when(wid == NW - 1)
        def _():
            do_window(ALIGNED, 64, True)

    one_table(t_item, tail_item, item_out, 0)
    one_table(t_user, tail_user, user_out, 1)


@functools.cache
def _sc_gather_kernel():
    mesh = plsc.VectorSubcoreMesh(core_axis_name="c", subcore_axis_name="s")
    return pl.kernel(
        _sc_gather_body,
        mesh=mesh,
        compiler_params=pltpu.CompilerParams(needs_layout_passes=False),
        out_type=[
            jax.ShapeDtypeStruct((BATCH + 128, 4 * EMB), jnp.float32),
            jax.ShapeDtypeStruct((BATCH + 128, 4 * EMB), jnp.float32),
        ],
        scratch_types=[
            pltpu.VMEM((BATCH // 128, 128), jnp.int32),
            pltpu.VMEM((1664,), jnp.int32),
            pltpu.VMEM((1664,), jnp.int32),
            pltpu.VMEM((2 * CAP + LANES,), jnp.int32),
            pltpu.VMEM((128, 128), jnp.float32),
            pltpu.VMEM((1, 128), jnp.int32),
            pltpu.VMEM((EMB, WIN), jnp.float32),
            pltpu.VMEM((EMB, 64), jnp.float32),
            pltpu.SemaphoreType.DMA,
            pltpu.SemaphoreType.DMA,
        ],
    )


BLK = 2048


def _mlp_body(item_ref, user_ref, w1a_ref, w1b_ref, b1_ref,
              w2_ref, b2_ref, w3_ref, b3_ref, w4_ref, b4_ref, out_ref):
    xi = item_ref[:, 0:EMB]
    xu = user_ref[:, 0:EMB]
    h = xi @ w1a_ref[...] + xu @ w1b_ref[...]
    h = jax.nn.relu(h + b1_ref[...])
    h = jax.nn.relu(h @ w2_ref[...] + b2_ref[...])
    h = jax.nn.relu(h @ w3_ref[...] + b3_ref[...])
    o = h @ w4_ref[...] + b4_ref[...]
    out_ref[...] = jax.nn.sigmoid(o)


def _mlp(item_rows, user_rows, w1a, w1b, b1, w2, b2, w3, b3, w4, b4):
    grid = (BATCH // BLK,)
    full = lambda shape: pl.BlockSpec(shape, lambda i: (0, 0))
    return pl.pallas_call(
        _mlp_body,
        grid=grid,
        in_specs=[
            pl.BlockSpec((BLK, 4 * EMB), lambda i: (i, 0)),
            pl.BlockSpec((BLK, 4 * EMB), lambda i: (i, 0)),
            full(w1a.shape), full(w1b.shape), full(b1.shape),
            full(w2.shape), full(b2.shape),
            full(w3.shape), full(b3.shape),
            full(w4.shape), full(b4.shape),
        ],
        out_specs=pl.BlockSpec((BLK, 1), lambda i: (i, 0)),
        out_shape=jax.ShapeDtypeStruct((BATCH, 1), jnp.float32),
    )(item_rows, user_rows, w1a, w1b, b1, w2, b2, w3, b3, w4, b4)


def kernel(item_input, user_input, emb_item, emb_user,
           W1, b1, W2, b2, W3, b3, W4, b4):
    idx_all = jnp.stack([item_input.astype(jnp.int32),
                         user_input.astype(jnp.int32)]).reshape(
                             2, BATCH // 128, 128)
    item_g, user_g = _sc_gather_kernel()(
        emb_item.T, emb_user.T,
        emb_item.T[:, ALIGNED:], emb_user.T[:, ALIGNED:], idx_all)
    out = _mlp(item_g, user_g,
               W1[:, :EMB].T, W1[:, EMB:].T, b1.reshape(1, -1),
               W2.T, b2.reshape(1, -1),
               W3.T, b3.reshape(1, -1),
               W4.T, b4.reshape(1, 1))
    return out.reshape(BATCH)


# speedup vs baseline: 3.8673x; 2.8109x over previous
import functools

import jax
import jax.numpy as jnp
from jax import lax
from jax.experimental import pallas as pl
from jax.experimental.pallas import tpu as pltpu
from jax.experimental.pallas import tpu_sc as plsc

BATCH = 16384
EMB = 32
NC = 2
NS = 16
NW = NC * NS
NIDX = 1000000
ALIGNED = 999936
BASE_TC = 244
WIN = 1024
NWIN = 30
CAP = 160
LCAP = 2048
DUMP = BATCH
LANES = 16


def _sc_gather_body(t_item, t_user, tail_item, tail_user, idx_all,
                    item_out, user_out,
                    idx_v, li_v, lp_v, wi_v, rows_v, pos_v, win_v,
                    tail_v, sem, ssem):
    wid = lax.axis_index("s") * NC + lax.axis_index("c")
    startc = BASE_TC * wid + jnp.minimum(wid, 4)
    start = startc * 128
    extra = (wid < 4).astype(jnp.int32)
    end_aligned = (startc + BASE_TC + extra) * 128
    scan_end = jnp.where(wid == NW - 1, NIDX, end_aligned)
    iota = lax.iota(jnp.int32, LANES)

    def one_table(tbl, tail_tbl, out_hbm, tbl_pos):
        pltpu.sync_copy(idx_all.at[tbl_pos], idx_v)
        pltpu.sync_copy(tail_tbl, tail_v)

        def scan(j, cnt):
            r = j // 8
            l = (j % 8) * LANES
            raw = idx_v[r, pl.ds(l, LANES)]
            m = (raw >= start) & (raw < scan_end)
            plsc.store_compressed(li_v.at[pl.ds(cnt, LANES)], raw, mask=m)
            pos = j * LANES + iota
            plsc.store_compressed(lp_v.at[pl.ds(cnt, LANES)], pos, mask=m)
            return cnt + plsc.all_reduce_population_count(m)[0]

        count = lax.fori_loop(0, BATCH // LANES, scan, 0)
        ntrip = (count + LANES - 1) // LANES

        def do_window(c0, wsize, from_tail):
            if not from_tail:
                wcp = pltpu.async_copy(tbl.at[:, pl.ds(c0, wsize)],
                                       win_v.at[:, pl.ds(0, wsize)], sem)

            def collect(q, wcnt):
                il = li_v[pl.ds(q * LANES, LANES)]
                ip = lp_v[pl.ds(q * LANES, LANES)]
                valid = (q * LANES + iota) < count
                m = valid & (il >= c0) & (il < c0 + wsize)
                plsc.store_compressed(wi_v.at[pl.ds(wcnt, LANES)], il - c0,
                                      mask=m)
                plsc.store_compressed(wi_v.at[pl.ds(CAP + wcnt, LANES)], ip,
                                      mask=m)
                return wcnt + plsc.all_reduce_population_count(m)[0]

            wcount = lax.fori_loop(0, ntrip, collect, 0)
            if not from_tail:
                wcp.wait()

            @pl.when(wcount > 0)
            def _():
                src_v = tail_v if from_tail else win_v

                def extract(q, _):
                    cols = wi_v[pl.ds(q * LANES, LANES)]
                    for k in range(LANES):
                        col = jnp.clip(cols[k], 0, wsize - 1)
                        csp = jnp.full((LANES,), col, jnp.int32)
                        v0 = plsc.load_gather(src_v, [iota, csp])
                        v1 = plsc.load_gather(src_v, [iota + LANES, csp])
                        m = q * LANES + k
                        rows_v[m, pl.ds(0, LANES)] = v0
                        rows_v[m, pl.ds(LANES, LANES)] = v1
                    pv = wi_v[pl.ds(CAP + q * LANES, LANES)]
                    pv = jnp.where(q * LANES + iota < wcount, pv,
                                   DUMP + ((wid * 4 + q) & 127))
                    pos_v[q, pl.ds(0, LANES)] = pv
                    pltpu.async_copy(rows_v.at[pl.ds(q * LANES, LANES)],
                                     out_hbm.at[pos_v.at[q]], ssem)
                    return 0

                wtrip = (wcount + LANES - 1) // LANES
                lax.fori_loop(0, wtrip, extract, 0)

                def drain(q, _):
                    pltpu.make_async_copy(
                        tbl.at[pl.ds(0, LANES), pl.ds(0, 128)],
                        rows_v.at[pl.ds(q * LANES, LANES)], ssem).wait()
                    return 0

                lax.fori_loop(0, wtrip, drain, 0)

        def window_loop(w, _):
            do_window(start + w * WIN, WIN, False)
            return 0

        lax.fori_loop(0, NWIN, window_loop, 0)
        do_window(start + NWIN * WIN, 512, False)

        @pl.when(wid < 4)
        def _():
            do_window(end_aligned - 128, 128, False)

        @pl.when(wid == NW - 1)
        def _():
            do_window(ALIGNED, 64, True)

    one_table(t_item, tail_item, item_out, 0)
    one_table(t_user, tail_user, user_out, 1)


@functools.cache
def _sc_gather_kernel():
    mesh = plsc.VectorSubcoreMesh(core_axis_name="c", subcore_axis_name="s")
    return pl.kernel(
        _sc_gather_body,
        mesh=mesh,
        compiler_params=pltpu.CompilerParams(needs_layout_passes=False),
        out_type=[
            jax.ShapeDtypeStruct((BATCH + 128, 4 * EMB), jnp.float32),
            jax.ShapeDtypeStruct((BATCH + 128, 4 * EMB), jnp.float32),
        ],
        scratch_types=[
            pltpu.VMEM((BATCH // 128, 128), jnp.int32),
            pltpu.VMEM((LCAP,), jnp.int32),
            pltpu.VMEM((LCAP,), jnp.int32),
            pltpu.VMEM((2 * CAP + LANES,), jnp.int32),
            pltpu.VMEM((CAP, 128), jnp.float32),
            pltpu.VMEM((CAP // LANES, LANES), jnp.int32),
            pltpu.VMEM((EMB, WIN), jnp.float32),
            pltpu.VMEM((EMB, 64), jnp.float32),
            pltpu.SemaphoreType.DMA,
            pltpu.SemaphoreType.DMA,
        ],
    )


BLK = 2048


def _mlp_body(item_ref, user_ref, w1a_ref, w1b_ref, b1_ref,
              w2_ref, b2_ref, w3_ref, b3_ref, w4_ref, b4_ref, out_ref):
    xi = item_ref[:, 0:EMB]
    xu = user_ref[:, 0:EMB]
    h = xi @ w1a_ref[...] + xu @ w1b_ref[...]
    h = jax.nn.relu(h + b1_ref[...])
    h = jax.nn.relu(h @ w2_ref[...] + b2_ref[...])
    h = jax.nn.relu(h @ w3_ref[...] + b3_ref[...])
    o = h @ w4_ref[...] + b4_ref[...]
    out_ref[...] = jax.nn.sigmoid(o)


def _mlp(item_rows, user_rows, w1a, w1b, b1, w2, b2, w3, b3, w4, b4):
    grid = (BATCH // BLK,)
    full = lambda shape: pl.BlockSpec(shape, lambda i: (0, 0))
    return pl.pallas_call(
        _mlp_body,
        grid=grid,
        in_specs=[
            pl.BlockSpec((BLK, 4 * EMB), lambda i: (i, 0)),
            pl.BlockSpec((BLK, 4 * EMB), lambda i: (i, 0)),
            full(w1a.shape), full(w1b.shape), full(b1.shape),
            full(w2.shape), full(b2.shape),
            full(w3.shape), full(b3.shape),
            full(w4.shape), full(b4.shape),
        ],
        out_specs=pl.BlockSpec((BLK, 1), lambda i: (i, 0)),
        out_shape=jax.ShapeDtypeStruct((BATCH, 1), jnp.float32),
    )(item_rows, user_rows, w1a, w1b, b1, w2, b2, w3, b3, w4, b4)


def kernel(item_input, user_input, emb_item, emb_user,
           W1, b1, W2, b2, W3, b3, W4, b4):
    idx_all = jnp.stack([item_input.astype(jnp.int32),
                         user_input.astype(jnp.int32)]).reshape(
                             2, BATCH // 128, 128)
    item_g, user_g = _sc_gather_kernel()(
        emb_item.T, emb_user.T,
        emb_item.T[:, ALIGNED:], emb_user.T[:, ALIGNED:], idx_all)
    out = _mlp(item_g, user_g,
               W1[:, :EMB].T, W1[:, EMB:].T, b1.reshape(1, -1),
               W2.T, b2.reshape(1, -1),
               W3.T, b3.reshape(1, -1),
               W4.T, b4.reshape(1, 1))
    return out.reshape(BATCH)


# speedup vs baseline: 4.2757x; 1.1056x over previous
import functools

import jax
import jax.numpy as jnp
from jax import lax
from jax.experimental import pallas as pl
from jax.experimental.pallas import tpu as pltpu
from jax.experimental.pallas import tpu_sc as plsc

BATCH = 16384
EMB = 32
NC = 2
NS = 16
NW = NC * NS
NIDX = 1000000
ALIGNED = 999936
BASE_TC = 244
WIN = 1024
NWIN = 30
CAP = 160
LCAP = 2048
DUMP = BATCH
LANES = 16


def _sc_gather_body(t_item, t_user, tail_item, tail_user, idx_all,
                    item_out, user_out,
                    idx_v, li_v, lp_v, wi_v, rows_v, pos_v, win_v,
                    win_b, tail_v, sem, semb, ssem):
    wid = lax.axis_index("s") * NC + lax.axis_index("c")
    startc = BASE_TC * wid + jnp.minimum(wid, 4)
    start = startc * 128
    extra = (wid < 4).astype(jnp.int32)
    end_aligned = (startc + BASE_TC + extra) * 128
    scan_end = jnp.where(wid == NW - 1, NIDX, end_aligned)
    iota = lax.iota(jnp.int32, LANES)

    def one_table(tbl, tail_tbl, out_hbm, tbl_pos):
        pltpu.sync_copy(idx_all.at[tbl_pos], idx_v)
        pltpu.sync_copy(tail_tbl, tail_v)

        def scan(j, cnt):
            r = j // 8
            l = (j % 8) * LANES
            raw = idx_v[r, pl.ds(l, LANES)]
            m = (raw >= start) & (raw < scan_end)
            plsc.store_compressed(li_v.at[pl.ds(cnt, LANES)], raw, mask=m)
            pos = j * LANES + iota
            plsc.store_compressed(lp_v.at[pl.ds(cnt, LANES)], pos, mask=m)
            return cnt + plsc.all_reduce_population_count(m)[0]

        count = lax.fori_loop(0, BATCH // LANES, scan, 0)
        ntrip = (count + LANES - 1) // LANES

        def collect(c0, wsize):
            def step(q, wcnt):
                il = li_v[pl.ds(q * LANES, LANES)]
                ip = lp_v[pl.ds(q * LANES, LANES)]
                valid = (q * LANES + iota) < count
                m = valid & (il >= c0) & (il < c0 + wsize)
                plsc.store_compressed(wi_v.at[pl.ds(wcnt, LANES)], il - c0,
                                      mask=m)
                plsc.store_compressed(wi_v.at[pl.ds(CAP + wcnt, LANES)], ip,
                                      mask=m)
                return wcnt + plsc.all_reduce_population_count(m)[0]

            return lax.fori_loop(0, ntrip, step, 0)

        def extract_scatter(src_v, wcount, wsize):
            @pl.when(wcount > 0)
            def _():
                def extract(q, _):
                    cols = wi_v[pl.ds(q * LANES, LANES)]
                    for k in range(LANES):
                        col = jnp.clip(cols[k], 0, wsize - 1)
                        csp = jnp.full((LANES,), col, jnp.int32)
                        v0 = plsc.load_gather(src_v, [iota, csp])
                        v1 = plsc.load_gather(src_v, [iota + LANES, csp])
                        m = q * LANES + k
                        rows_v[m, pl.ds(0, LANES)] = v0
                        rows_v[m, pl.ds(LANES, LANES)] = v1
                    pv = wi_v[pl.ds(CAP + q * LANES, LANES)]
                    pv = jnp.where(q * LANES + iota < wcount, pv,
                                   DUMP + ((wid * 4 + q) & 127))
                    pos_v[q, pl.ds(0, LANES)] = pv
                    pltpu.async_copy(rows_v.at[pl.ds(q * LANES, LANES)],
                                     out_hbm.at[pos_v.at[q]], ssem)
                    return 0

                wtrip = (wcount + LANES - 1) // LANES
                lax.fori_loop(0, wtrip, extract, 0)

                def drain(q, _):
                    pltpu.make_async_copy(
                        tbl.at[pl.ds(0, LANES), pl.ds(0, 128)],
                        rows_v.at[pl.ds(q * LANES, LANES)], ssem).wait()
                    return 0

                lax.fori_loop(0, wtrip, drain, 0)

        pltpu.async_copy(tbl.at[:, pl.ds(start, WIN)], win_v, sem)

        def window_loop(w, _):
            c0 = start + w * WIN
            wcount = collect(c0, WIN)

            def run(buf, s, obuf, os):
                @pl.when(w + 1 < NWIN)
                def _():
                    pltpu.async_copy(
                        tbl.at[:, pl.ds(c0 + WIN, WIN)], obuf, os)
                pltpu.make_async_copy(
                    tbl.at[:, pl.ds(0, WIN)], buf, s).wait()
                extract_scatter(buf, wcount, WIN)

            @pl.when(w % 2 == 0)
            def _():
                run(win_v, sem, win_b, semb)

            @pl.when(w % 2 == 1)
            def _():
                run(win_b, semb, win_v, sem)

            return 0

        lax.fori_loop(0, NWIN, window_loop, 0)

        def do_small(c0, wsize, src_v, stream):
            wcount = collect(c0, wsize)
            if stream:
                pltpu.sync_copy(tbl.at[:, pl.ds(c0, wsize)],
                                src_v.at[:, pl.ds(0, wsize)])
            extract_scatter(src_v, wcount, wsize)

        do_small(start + NWIN * WIN, 512, win_v, True)

        @pl.when(wid < 4)
        def _():
            do_small(end_aligned - 128, 128, win_v, True)

        @pl.when(wid == NW - 1)
        def _():
            do_small(ALIGNED, 64, tail_v, False)

    one_table(t_item, tail_item, item_out, 0)
    one_table(t_user, tail_user, user_out, 1)


@functools.cache
def _sc_gather_kernel():
    mesh = plsc.VectorSubcoreMesh(core_axis_name="c", subcore_axis_name="s")
    return pl.kernel(
        _sc_gather_body,
        mesh=mesh,
        compiler_params=pltpu.CompilerParams(needs_layout_passes=False),
        out_type=[
            jax.ShapeDtypeStruct((BATCH + 128, 4 * EMB), jnp.float32),
            jax.ShapeDtypeStruct((BATCH + 128, 4 * EMB), jnp.float32),
        ],
        scratch_types=[
            pltpu.VMEM((BATCH // 128, 128), jnp.int32),
            pltpu.VMEM((LCAP,), jnp.int32),
            pltpu.VMEM((LCAP,), jnp.int32),
            pltpu.VMEM((2 * CAP + LANES,), jnp.int32),
            pltpu.VMEM((CAP, 128), jnp.float32),
            pltpu.VMEM((CAP // LANES, LANES), jnp.int32),
            pltpu.VMEM((EMB, WIN), jnp.float32),
            pltpu.VMEM((EMB, WIN), jnp.float32),
            pltpu.VMEM((EMB, 64), jnp.float32),
            pltpu.SemaphoreType.DMA,
            pltpu.SemaphoreType.DMA,
            pltpu.SemaphoreType.DMA,
        ],
    )


BLK = 2048


def _mlp_body(item_ref, user_ref, w1a_ref, w1b_ref, b1_ref,
              w2_ref, b2_ref, w3_ref, b3_ref, w4_ref, b4_ref, out_ref):
    xi = item_ref[:, 0:EMB]
    xu = user_ref[:, 0:EMB]
    h = xi @ w1a_ref[...] + xu @ w1b_ref[...]
    h = jax.nn.relu(h + b1_ref[...])
    h = jax.nn.relu(h @ w2_ref[...] + b2_ref[...])
    h = jax.nn.relu(h @ w3_ref[...] + b3_ref[...])
    o = h @ w4_ref[...] + b4_ref[...]
    out_ref[...] = jax.nn.sigmoid(o)


def _mlp(item_rows, user_rows, w1a, w1b, b1, w2, b2, w3, b3, w4, b4):
    grid = (BATCH // BLK,)
    full = lambda shape: pl.BlockSpec(shape, lambda i: (0, 0))
    return pl.pallas_call(
        _mlp_body,
        grid=grid,
        in_specs=[
            pl.BlockSpec((BLK, 4 * EMB), lambda i: (i, 0)),
            pl.BlockSpec((BLK, 4 * EMB), lambda i: (i, 0)),
            full(w1a.shape), full(w1b.shape), full(b1.shape),
            full(w2.shape), full(b2.shape),
            full(w3.shape), full(b3.shape),
            full(w4.shape), full(b4.shape),
        ],
        out_specs=pl.BlockSpec((BLK, 1), lambda i: (i, 0)),
        out_shape=jax.ShapeDtypeStruct((BATCH, 1), jnp.float32),
    )(item_rows, user_rows, w1a, w1b, b1, w2, b2, w3, b3, w4, b4)


def kernel(item_input, user_input, emb_item, emb_user,
           W1, b1, W2, b2, W3, b3, W4, b4):
    idx_all = jnp.stack([item_input.astype(jnp.int32),
                         user_input.astype(jnp.int32)]).reshape(
                             2, BATCH // 128, 128)
    item_g, user_g = _sc_gather_kernel()(
        emb_item.T, emb_user.T,
        emb_item.T[:, ALIGNED:], emb_user.T[:, ALIGNED:], idx_all)
    out = _mlp(item_g, user_g,
               W1[:, :EMB].T, W1[:, EMB:].T, b1.reshape(1, -1),
               W2.T, b2.reshape(1, -1),
               W3.T, b3.reshape(1, -1),
               W4.T, b4.reshape(1, 1))
    return out.reshape(BATCH)
